# Initial kernel scaffold; baseline (speedup 1.0000x reference)
import jax
import jax.numpy as jnp
from jax.experimental import pallas as pl


def kernel(user_ids, item_ids, global_bias, linear_weights, embeddings):
    raise NotImplementedError("write your pallas kernel here")


# speedup vs baseline: 1.2635x; 1.2635x over previous
import functools

import jax
import jax.numpy as jnp
from jax import lax
from jax.experimental import pallas as pl
from jax.experimental.pallas import tpu as pltpu
from jax.experimental.pallas import tpu_sc as plsc

_NUM_USERS = 100000
_LANES = 16
_IDX_CHUNK = 128


@functools.cache
def _build_fm_kernel(batch: int, num_features: int, embed_dim: int):
  info = plsc.get_sparse_core_info()
  num_workers = info.num_cores * info.num_subcores
  bpw = batch // num_workers
  assert batch % (8 * num_workers) == 0
  assert embed_dim % _LANES == 0
  n_chunks = bpw // _IDX_CHUNK
  n_groups = bpw // _LANES
  d_vecs = embed_dim // _LANES

  mesh = plsc.VectorSubcoreMesh(core_axis_name="c", subcore_axis_name="s")

  @functools.partial(
      pl.kernel,
      out_type=jax.ShapeDtypeStruct((batch,), jnp.float32),
      mesh=mesh,
      compiler_params=pltpu.CompilerParams(
          needs_layout_passes=False, use_tc_tiling_on_sc=False),
      scratch_types=[
          pltpu.VMEM((bpw,), jnp.int32),
          pltpu.VMEM((bpw,), jnp.int32),
          pltpu.VMEM((bpw, embed_dim), jnp.float32),
          pltpu.VMEM((bpw, embed_dim), jnp.float32),
          pltpu.VMEM((bpw,), jnp.float32),
          pltpu.VMEM((bpw,), jnp.float32),
          pltpu.VMEM((_LANES,), jnp.float32),
          pltpu.VMEM((_LANES * _LANES,), jnp.float32),
          pltpu.VMEM((bpw,), jnp.float32),
          pltpu.SemaphoreType.DMA,
      ],
  )
  def fm(uid_hbm, iid_hbm, bias_hbm, lw_hbm, emb_hbm, out_hbm,
         uidx_v, iidx_v, rows_u, rows_i, lwu_v, lwi_v, bias_v, sums_v,
         out_v, sem):
    wid = lax.axis_index("s") * info.num_cores + lax.axis_index("c")
    base = wid * bpw

    pltpu.sync_copy(uid_hbm.at[pl.ds(base, bpw)], uidx_v)
    pltpu.sync_copy(iid_hbm.at[pl.ds(base, bpw)], iidx_v)
    pltpu.sync_copy(bias_hbm, bias_v)

    for k in range(bpw // _LANES):
      sl = pl.ds(k * _LANES, _LANES)
      iidx_v[sl] = iidx_v[sl] + _NUM_USERS

    copies = []
    for c in range(n_chunks):
      sl = pl.ds(c * _IDX_CHUNK, _IDX_CHUNK)
      copies.append(
          pltpu.async_copy(emb_hbm.at[uidx_v.at[sl]], rows_u.at[sl], sem))
      copies.append(
          pltpu.async_copy(emb_hbm.at[iidx_v.at[sl]], rows_i.at[sl], sem))
      copies.append(
          pltpu.async_copy(lw_hbm.at[uidx_v.at[sl]], lwu_v.at[sl], sem))
      copies.append(
          pltpu.async_copy(lw_hbm.at[iidx_v.at[sl]], lwi_v.at[sl], sem))
    for cp in copies:
      cp.wait()

    lane = lax.iota(jnp.int32, _LANES)
    bias_vec = bias_v[pl.ds(0, _LANES)]

    def group_body(g, _):
      for j in range(_LANES):
        r = g * _LANES + j
        s = None
        for k in range(d_vecs):
          dsl = pl.ds(k * _LANES, _LANES)
          p = rows_u[r, dsl] * rows_i[r, dsl]
          s = p if s is None else s + p
        sums_v[pl.ds(j * _LANES, _LANES)] = s
      gsl = pl.ds(g * _LANES, _LANES)
      acc = bias_vec + lwu_v[gsl] + lwi_v[gsl]
      row_base = lane * _LANES
      for c in range(_LANES):
        acc = acc + plsc.load_gather(sums_v, [row_base + c])
      out_v[gsl] = acc
      return _

    lax.fori_loop(0, n_groups, group_body, None)
    pltpu.sync_copy(out_v, out_hbm.at[pl.ds(base, bpw)])

  return fm


def kernel(user_ids, item_ids, global_bias, linear_weights, embeddings):
  batch = user_ids.shape[0]
  num_features, embed_dim = embeddings.shape
  lw_flat = linear_weights.reshape(num_features)
  bias16 = jnp.broadcast_to(global_bias.astype(jnp.float32), (_LANES,))
  fm = _build_fm_kernel(batch, num_features, embed_dim)
  out = fm(user_ids.astype(jnp.int32), item_ids.astype(jnp.int32),
           bias16, lw_flat, embeddings)
  return out.reshape(batch, 1)
